# baseline (device time: 29240 ns/iter reference)
import jax
import jax.numpy as jnp
from jax import lax
from jax.experimental import pallas as pl
from jax.experimental.pallas import tpu as pltpu

N_CHUNKS = 8


def kernel(x, W):
    T, D = x.shape
    D2, V_local = W.shape
    V = 2 * V_local
    QCOLS = V_local // 4
    CH = QCOLS // 2

    def body(x_ref, w_hbm, out_hbm, w_ref, l_ref, p_ref, o_ref, s_ref,
             sp_ref, wsems, osems, send_sems, recv_sems, ssem_s, ssem_r):
        my_x = lax.axis_index("x")
        my_y = lax.axis_index("y")
        my_z = lax.axis_index("z")
        k_own = 2 * my_y + my_z
        k_diag = 3 - k_own
        q2 = jnp.where((k_own == 0) | (k_own == 3), 1, 0)
        q3 = 3 - q2
        k_y = k_own ^ 2
        k_z = k_own ^ 1
        x_nbr = (1 - my_x, my_y, my_z)
        y_nbr = (my_x, 1 - my_y, my_z)
        z_nbr = (my_x, my_y, 1 - my_z)
        qs = [k_own, k_diag, q2, q3]

        wdmas = [
            pltpu.make_async_copy(
                w_hbm.at[:, pl.ds(qs[i] * QCOLS, QCOLS)],
                w_ref.at[i], wsems.at[i],
            )
            for i in range(4)
        ]
        for d in wdmas:
            d.start()

        barrier = pltpu.get_barrier_semaphore()
        for nbr in (x_nbr, y_nbr, z_nbr):
            pl.semaphore_signal(
                barrier, inc=1, device_id=nbr,
                device_id_type=pl.DeviceIdType.MESH,
            )
        pl.semaphore_wait(barrier, 3)

        xb = x_ref[...].astype(jnp.bfloat16)

        def rdma(src, dst, sem_i, dev):
            return pltpu.make_async_remote_copy(
                src_ref=src, dst_ref=dst,
                send_sem=send_sems.at[sem_i], recv_sem=recv_sems.at[sem_i],
                device_id=dev, device_id_type=pl.DeviceIdType.MESH,
            )

        for i in range(4):
            wdmas[i].wait()
            wq = w_ref[i].astype(jnp.bfloat16)
            lg = lax.dot_general(
                xb, wq, (((1,), (0,)), ((), ())),
                preferred_element_type=jnp.float32,
            ).astype(jnp.bfloat16)
            l_ref[2 * i] = lg[:, :CH]
            l_ref[2 * i + 1] = lg[:, CH:]
            if i == 0:
                x1a = rdma(l_ref.at[0], p_ref.at[2 * k_own], 0, x_nbr)
                x1a.start()
                x1b = rdma(l_ref.at[1], p_ref.at[2 * k_own + 1], 1, x_nbr)
                x1b.start()

        s = jnp.zeros((T, 1), jnp.float32)
        for i in range(4):
            lg = jnp.concatenate(
                [l_ref[2 * i], l_ref[2 * i + 1]], axis=1
            ).astype(jnp.float32)
            s = s + jnp.sum(jnp.exp(lg), axis=1, keepdims=True)
        s_ref[...] = s
        sdma = pltpu.make_async_remote_copy(
            src_ref=s_ref, dst_ref=sp_ref, send_sem=ssem_s,
            recv_sem=ssem_r, device_id=x_nbr,
            device_id_type=pl.DeviceIdType.MESH,
        )
        sdma.start()

        x2a = rdma(l_ref.at[2], p_ref.at[2 * k_diag], 2, x_nbr)
        x2a.start()
        x2b = rdma(l_ref.at[3], p_ref.at[2 * k_diag + 1], 3, x_nbr)
        x2b.start()

        x1a.wait_recv()
        y1a = rdma(p_ref.at[2 * k_own], p_ref.at[2 * k_own], 4, y_nbr)
        y1a.start()
        z1a = rdma(p_ref.at[2 * k_own], p_ref.at[2 * k_own], 6, z_nbr)
        z1a.start()
        x1b.wait_recv()
        y1b = rdma(p_ref.at[2 * k_own + 1], p_ref.at[2 * k_own + 1], 5, y_nbr)
        y1b.start()
        z1b = rdma(p_ref.at[2 * k_own + 1], p_ref.at[2 * k_own + 1], 7, z_nbr)
        z1b.start()

        sdma.wait_recv()
        inv = 1.0 / (s + sp_ref[...])

        out_dmas = []

        def flush(e, col, slot):
            o_ref[slot] = e
            d = pltpu.make_async_copy(
                o_ref.at[slot], out_hbm.at[:, pl.ds(col, CH)], osems.at[slot]
            )
            d.start()
            out_dmas.append(d)

        loc = my_x * V_local
        for i in range(4):
            for h in range(2):
                e = jnp.exp(l_ref[2 * i + h].astype(jnp.float32)) * inv
                flush(e, loc + qs[i] * QCOLS + h * CH, 2 * i + h)

        rem = (1 - my_x) * V_local

        def flush_peer(g, slot):
            e = jnp.exp(p_ref[g].astype(jnp.float32)) * inv
            flush(e, rem + g * CH, slot)

        flush_peer(2 * k_own, 8)
        flush_peer(2 * k_own + 1, 9)
        x2a.wait_recv()
        flush_peer(2 * k_diag, 10)
        x2b.wait_recv()
        flush_peer(2 * k_diag + 1, 11)

        for slot, (sem_i, g, nbr) in enumerate(
            (
                (4, 2 * k_y, y_nbr),
                (5, 2 * k_y + 1, y_nbr),
                (6, 2 * k_z, z_nbr),
                (7, 2 * k_z + 1, z_nbr),
            ),
            start=12,
        ):
            rdma(p_ref.at[g], p_ref.at[g], sem_i, nbr).wait_recv()
            flush_peer(g, slot)

        for r in (x1a, x1b, x2a, x2b, y1a, y1b, z1a, z1b, sdma):
            r.wait_send()
        for d in out_dmas:
            d.wait()

    return pl.pallas_call(
        body,
        out_shape=jax.ShapeDtypeStruct((T, V), jnp.float32),
        in_specs=[
            pl.BlockSpec(memory_space=pltpu.VMEM),
            pl.BlockSpec(memory_space=pltpu.MemorySpace.HBM),
        ],
        out_specs=pl.BlockSpec(memory_space=pltpu.MemorySpace.HBM),
        scratch_shapes=[
            pltpu.VMEM((4, D, QCOLS), jnp.float32),
            pltpu.VMEM((N_CHUNKS, T, CH), jnp.bfloat16),
            pltpu.VMEM((N_CHUNKS, T, CH), jnp.bfloat16),
            pltpu.VMEM((2 * N_CHUNKS, T, CH), jnp.float32),
            pltpu.VMEM((T, 1), jnp.float32),
            pltpu.VMEM((T, 1), jnp.float32),
            pltpu.SemaphoreType.DMA((4,)),
            pltpu.SemaphoreType.DMA((2 * N_CHUNKS,)),
            pltpu.SemaphoreType.DMA((N_CHUNKS,)),
            pltpu.SemaphoreType.DMA((N_CHUNKS,)),
            pltpu.SemaphoreType.DMA,
            pltpu.SemaphoreType.DMA,
        ],
        compiler_params=pltpu.CompilerParams(collective_id=0),
    )(x, W)


# device time: 28687 ns/iter; 1.0193x vs baseline; 1.0193x over previous
import contextlib
import os

import jax
import jax.numpy as jnp
from jax import lax
from jax.experimental import pallas as pl
from jax.experimental.pallas import tpu as pltpu

N_CHUNKS = 8

_PROF = os.environ.get("KPROF") == "1"


def _scope(name):
    return jax.named_scope(name) if _PROF else contextlib.nullcontext()


def kernel(x, W):
    T, D = x.shape
    D2, V_local = W.shape
    V = 2 * V_local
    QCOLS = V_local // 4
    CH = QCOLS // 2

    def body(x_ref, w_hbm, out_hbm, w_ref, l_ref, p_ref, o_ref, s_ref,
             sp_ref, wsems, osems, send_sems, recv_sems, ssem_s, ssem_r):
        my_x = lax.axis_index("x")
        my_y = lax.axis_index("y")
        my_z = lax.axis_index("z")
        k_own = 2 * my_y + my_z
        k_diag = 3 - k_own
        q2 = jnp.where((k_own == 0) | (k_own == 3), 1, 0)
        q3 = 3 - q2
        k_y = k_own ^ 2
        k_z = k_own ^ 1
        x_nbr = (1 - my_x, my_y, my_z)
        y_nbr = (my_x, 1 - my_y, my_z)
        z_nbr = (my_x, my_y, 1 - my_z)
        qs = [k_own, k_diag, q2, q3]

        with _scope("wload_barrier"):
            wdmas = [
                pltpu.make_async_copy(
                    w_hbm.at[:, pl.ds(qs[i] * QCOLS, QCOLS)],
                    w_ref.at[i], wsems.at[i],
                )
                for i in range(4)
            ]
            for d in wdmas:
                d.start()

            barrier = pltpu.get_barrier_semaphore()
            for nbr in (x_nbr, y_nbr, z_nbr):
                pl.semaphore_signal(
                    barrier, inc=1, device_id=nbr,
                    device_id_type=pl.DeviceIdType.MESH,
                )

        def rdma(src, dst, sem_i, dev):
            return pltpu.make_async_remote_copy(
                src_ref=src, dst_ref=dst,
                send_sem=send_sems.at[sem_i], recv_sem=recv_sems.at[sem_i],
                device_id=dev, device_id_type=pl.DeviceIdType.MESH,
            )

        with _scope("gemm"):
            xb = x_ref[...].astype(jnp.bfloat16)
            for i in range(4):
                wdmas[i].wait()
                wq = w_ref[i].astype(jnp.bfloat16)
                lg = lax.dot_general(
                    xb, wq, (((1,), (0,)), ((), ())),
                    preferred_element_type=jnp.float32,
                ).astype(jnp.bfloat16)
                l_ref[2 * i] = lg[:, :CH]
                l_ref[2 * i + 1] = lg[:, CH:]
                if i == 0:
                    pl.semaphore_wait(barrier, 3)
                    x1a = rdma(l_ref.at[0], p_ref.at[2 * k_own], 0, x_nbr)
                    x1a.start()
                    x1b = rdma(l_ref.at[1], p_ref.at[2 * k_own + 1], 1, x_nbr)
                    x1b.start()

        with _scope("sumexp_send"):
            s = jnp.zeros((T, 1), jnp.float32)
            for i in range(4):
                lg = jnp.concatenate(
                    [l_ref[2 * i], l_ref[2 * i + 1]], axis=1
                ).astype(jnp.float32)
                s = s + jnp.sum(jnp.exp(lg), axis=1, keepdims=True)
            s_ref[...] = s
            sdma = pltpu.make_async_remote_copy(
                src_ref=s_ref, dst_ref=sp_ref, send_sem=ssem_s,
                recv_sem=ssem_r, device_id=x_nbr,
                device_id_type=pl.DeviceIdType.MESH,
            )
            sdma.start()

            x2a = rdma(l_ref.at[2], p_ref.at[2 * k_diag], 2, x_nbr)
            x2a.start()
            x2b = rdma(l_ref.at[3], p_ref.at[2 * k_diag + 1], 3, x_nbr)
            x2b.start()

        with _scope("fwd"):
            x1a.wait_recv()
            y1a = rdma(p_ref.at[2 * k_own], p_ref.at[2 * k_own], 4, y_nbr)
            y1a.start()
            z1a = rdma(p_ref.at[2 * k_own], p_ref.at[2 * k_own], 6, z_nbr)
            z1a.start()
            x1b.wait_recv()
            y1b = rdma(p_ref.at[2 * k_own + 1], p_ref.at[2 * k_own + 1],
                       5, y_nbr)
            y1b.start()
            z1b = rdma(p_ref.at[2 * k_own + 1], p_ref.at[2 * k_own + 1],
                       7, z_nbr)
            z1b.start()

        with _scope("inv"):
            sdma.wait_recv()
            inv = 1.0 / (s + sp_ref[...])

        out_dmas = []

        def flush(e, col, slot):
            o_ref[slot] = e
            d = pltpu.make_async_copy(
                o_ref.at[slot], out_hbm.at[:, pl.ds(col, CH)], osems.at[slot]
            )
            d.start()
            out_dmas.append(d)

        with _scope("flush_local"):
            loc = my_x * V_local
            for i in range(4):
                for h in range(2):
                    e = jnp.exp(l_ref[2 * i + h].astype(jnp.float32)) * inv
                    flush(e, loc + qs[i] * QCOLS + h * CH, 2 * i + h)

        rem = (1 - my_x) * V_local

        def flush_peer(g, slot):
            e = jnp.exp(p_ref[g].astype(jnp.float32)) * inv
            flush(e, rem + g * CH, slot)

        with _scope("flush_peer_x"):
            flush_peer(2 * k_own, 8)
            flush_peer(2 * k_own + 1, 9)
            x2a.wait_recv()
            flush_peer(2 * k_diag, 10)
            x2b.wait_recv()
            flush_peer(2 * k_diag + 1, 11)

        with _scope("flush_fwd"):
            for slot, (sem_i, g, nbr) in enumerate(
                (
                    (4, 2 * k_y, y_nbr),
                    (5, 2 * k_y + 1, y_nbr),
                    (6, 2 * k_z, z_nbr),
                    (7, 2 * k_z + 1, z_nbr),
                ),
                start=12,
            ):
                rdma(p_ref.at[g], p_ref.at[g], sem_i, nbr).wait_recv()
                flush_peer(g, slot)

        with _scope("drain"):
            for r in (x1a, x1b, x2a, x2b, y1a, y1b, z1a, z1b, sdma):
                r.wait_send()
            for d in out_dmas:
                d.wait()

    return pl.pallas_call(
        body,
        out_shape=jax.ShapeDtypeStruct((T, V), jnp.float32),
        in_specs=[
            pl.BlockSpec(memory_space=pltpu.VMEM),
            pl.BlockSpec(memory_space=pltpu.MemorySpace.HBM),
        ],
        out_specs=pl.BlockSpec(memory_space=pltpu.MemorySpace.HBM),
        scratch_shapes=[
            pltpu.VMEM((4, D, QCOLS), jnp.float32),
            pltpu.VMEM((N_CHUNKS, T, CH), jnp.bfloat16),
            pltpu.VMEM((N_CHUNKS, T, CH), jnp.bfloat16),
            pltpu.VMEM((2 * N_CHUNKS, T, CH), jnp.float32),
            pltpu.VMEM((T, 1), jnp.float32),
            pltpu.VMEM((T, 1), jnp.float32),
            pltpu.SemaphoreType.DMA((4,)),
            pltpu.SemaphoreType.DMA((2 * N_CHUNKS,)),
            pltpu.SemaphoreType.DMA((N_CHUNKS,)),
            pltpu.SemaphoreType.DMA((N_CHUNKS,)),
            pltpu.SemaphoreType.DMA,
            pltpu.SemaphoreType.DMA,
        ],
        compiler_params=pltpu.CompilerParams(collective_id=0),
    )(x, W)
